# baseline (speedup 1.0000x reference)
import functools

import jax
import jax.numpy as jnp
from jax import lax
from jax.experimental import pallas as pl
from jax.experimental.pallas import tpu as pltpu
from jax.experimental.pallas import tpu_sc as plsc

V1, V2, V3 = 20, 40, 40
D1, D2, D3 = 4, 4, 8
RR = 4
NROWS = V1 * V2 * V3
DREAL = D1 * D2 * D3
DOUT = 2 * DREAL
ROWS_PER_I = V2 * V3
GH = RR * D3
RFH = RR * D2 * D3


def _prep_branch(c0, c1, c2):
    a = c0[0]
    b = c1
    c = c2[..., 0]
    eye_f = jnp.eye(D2, dtype=jnp.float32)
    eye_h = jnp.eye(D3, dtype=jnp.float32)
    cmat = c.transpose(1, 0, 2).reshape(V3, GH)
    v = jnp.einsum('rjfg,hq->jghrfq', b, eye_h).reshape(V2, GH, RFH)
    w = jnp.einsum('idr,fp,hq->irfhdpq', a, eye_f, eye_h).reshape(V1, RFH, DREAL)
    return cmat, v, w


def _table_body(cmr, vr, cmi, vi, wr, wi, outr_ref, outi_ref, qr, qi):
    i = pl.program_id(0)

    @pl.when(i == 0)
    def _():
        for j in range(V2):
            qr[pl.ds(j * V3, V3), :] = jnp.dot(
                cmr[...], vr[j], preferred_element_type=jnp.float32)
            qi[pl.ds(j * V3, V3), :] = jnp.dot(
                cmi[...], vi[j], preferred_element_type=jnp.float32)

    outr_ref[...] = jnp.dot(qr[...], wr[0], preferred_element_type=jnp.float32)
    outi_ref[...] = jnp.dot(qi[...], wi[0], preferred_element_type=jnp.float32)


def _build_tables(cmr, vr, wr, cmi, vi, wi):
    return pl.pallas_call(
        _table_body,
        grid=(V1,),
        in_specs=[
            pl.BlockSpec((V3, GH), lambda i: (0, 0)),
            pl.BlockSpec((V2, GH, RFH), lambda i: (0, 0, 0)),
            pl.BlockSpec((V3, GH), lambda i: (0, 0)),
            pl.BlockSpec((V2, GH, RFH), lambda i: (0, 0, 0)),
            pl.BlockSpec((1, RFH, DREAL), lambda i: (i, 0, 0)),
            pl.BlockSpec((1, RFH, DREAL), lambda i: (i, 0, 0)),
        ],
        out_specs=[
            pl.BlockSpec((ROWS_PER_I, DREAL), lambda i: (i, 0)),
            pl.BlockSpec((ROWS_PER_I, DREAL), lambda i: (i, 0)),
        ],
        out_shape=[
            jax.ShapeDtypeStruct((NROWS, DREAL), jnp.float32),
            jax.ShapeDtypeStruct((NROWS, DREAL), jnp.float32),
        ],
        scratch_shapes=[
            pltpu.VMEM((ROWS_PER_I, RFH), jnp.float32),
            pltpu.VMEM((ROWS_PER_I, RFH), jnp.float32),
        ],
    )(cmr, vr, cmi, vi, wr, wi)


SPAD = 56
CHUNK = 112


def _make_gather(n_tokens):
    info = plsc.get_sparse_core_info()
    nw = info.num_cores * info.num_subcores
    per_w = n_tokens // nw
    n_chunks = per_w // CHUNK
    assert per_w * nw == n_tokens and n_chunks * CHUNK == per_w
    assert n_chunks % 2 == 0
    mesh = plsc.VectorSubcoreMesh(core_axis_name="c", subcore_axis_name="s")

    @functools.partial(
        pl.kernel,
        mesh=mesh,
        out_type=[
            jax.ShapeDtypeStruct((n_tokens, DREAL), jnp.float32),
            jax.ShapeDtypeStruct((n_tokens, DREAL), jnp.float32),
        ],
        scratch_types=[
            pltpu.VMEM((per_w,), jnp.int32),
            pltpu.VMEM((2, CHUNK, DREAL), jnp.float32),
            pltpu.VMEM((2, CHUNK, DREAL), jnp.float32),
            pltpu.SemaphoreType.DMA,
            pltpu.SemaphoreType.DMA,
            pltpu.SemaphoreType.DMA,
            pltpu.SemaphoreType.DMA,
        ],
    )
    def gather_k(ids_hbm, tr_hbm, ti_hbm, outr_hbm, outi_hbm,
                 idx_v, rowsr_v, rowsi_v, sg0, sg1, ss0, ss1):
        sem_g = (sg0, sg1)
        sem_s = (ss0, ss1)
        wid = lax.axis_index("s") * info.num_cores + lax.axis_index("c")
        base = pl.multiple_of(wid * per_w, per_w)
        pltpu.sync_copy(ids_hbm.at[pl.ds(base, per_w)], idx_v)

        def mod_slice(t, carry):
            off = pl.multiple_of(t * 16, 16)
            idx_v[pl.ds(off, 16)] = lax.rem(idx_v[pl.ds(off, 16)], NROWS)
            return carry

        def g_copies(g, b):
            off = pl.multiple_of(g * CHUNK, CHUNK)
            sl = idx_v.at[pl.ds(off, CHUNK)]
            return (pltpu.make_async_copy(tr_hbm.at[sl], rowsr_v.at[b], sem_g[b]),
                    pltpu.make_async_copy(ti_hbm.at[sl], rowsi_v.at[b], sem_g[b]))

        def s_copies(g, b):
            off = pl.multiple_of(g * CHUNK, CHUNK)
            dst = pl.ds(base + off, CHUNK)
            return (pltpu.make_async_copy(rowsr_v.at[b], outr_hbm.at[dst], sem_s[b]),
                    pltpu.make_async_copy(rowsi_v.at[b], outi_hbm.at[dst], sem_s[b]))

        def start(copies):
            for c in copies:
                c.start()

        def wait(copies):
            for c in copies:
                c.wait()

        lax.fori_loop(0, CHUNK // 16, mod_slice, 0)
        start(g_copies(0, 0))
        lax.fori_loop(CHUNK // 16, per_w // 16, mod_slice, 0)

        def body(t, carry):
            for b in range(2):
                g = t * 2 + b
                wait(g_copies(g, b))
                start(s_copies(g, b))

                @pl.when(g >= 1)
                def _():
                    wait(s_copies(g - 1, 1 - b))

                @pl.when(g + 1 < n_chunks)
                def _():
                    start(g_copies(g + 1, 1 - b))

            return carry

        lax.fori_loop(0, n_chunks // 2, body, 0)
        wait(s_copies(n_chunks - 1, (n_chunks - 1) % 2))

    return gather_k


BBLK = 8


def _retile_body(r_ref, i_ref, out_ref):
    seq = out_ref.shape[1]
    r = r_ref[...].reshape(BBLK, SPAD, DREAL)
    i = i_ref[...].reshape(BBLK, SPAD, DREAL)
    out_ref[..., :DREAL] = r[:, :seq, :]
    out_ref[..., DREAL:] = i[:, :seq, :]


def _retile(plane_r, plane_i, batch, seq):
    return pl.pallas_call(
        _retile_body,
        grid=(batch // BBLK,),
        in_specs=[
            pl.BlockSpec((BBLK * SPAD, DREAL), lambda g: (g, 0)),
            pl.BlockSpec((BBLK * SPAD, DREAL), lambda g: (g, 0)),
        ],
        out_specs=pl.BlockSpec((BBLK, seq, DOUT), lambda g: (g, 0, 0)),
        out_shape=jax.ShapeDtypeStruct((batch, seq, DOUT), jnp.float32),
    )(plane_r, plane_i)


def kernel(input_ids, cr0, cr1, cr2, ci0, ci1, ci2):
    B, S = input_ids.shape
    ids = jnp.pad(input_ids.astype(jnp.int32), ((0, 0), (0, SPAD - S)))
    ids = ids.reshape(-1)
    cmr, vr, wr = _prep_branch(cr0, cr1, cr2)
    cmi, vi, wi = _prep_branch(ci0, ci1, ci2)
    table_r, table_i = _build_tables(cmr, vr, wr, cmi, vi, wi)
    plane_r, plane_i = _make_gather(B * SPAD)(ids, table_r, table_i)
    return _retile(plane_r, plane_i, B, S)


# speedup vs baseline: 1.1089x; 1.1089x over previous
import functools

import jax
import jax.numpy as jnp
from jax import lax
from jax.experimental import pallas as pl
from jax.experimental.pallas import tpu as pltpu
from jax.experimental.pallas import tpu_sc as plsc

V1, V2, V3 = 20, 40, 40
D1, D2, D3 = 4, 4, 8
RR = 4
NROWS = V1 * V2 * V3
DREAL = D1 * D2 * D3
DOUT = 2 * DREAL
ROWS_PER_I = V2 * V3
GH = RR * D3
RFH = RR * D2 * D3


def _prep_branch(c0, c1, c2):
    a = c0[0]
    b = c1
    c = c2[..., 0]
    eye_f = jnp.eye(D2, dtype=jnp.float32)
    eye_h = jnp.eye(D3, dtype=jnp.float32)
    cmat = c.transpose(1, 0, 2).reshape(V3, GH)
    v = jnp.einsum('rjfg,hq->jghrfq', b, eye_h).reshape(V2, GH, RFH)
    w = jnp.einsum('idr,fp,hq->irfhdpq', a, eye_f, eye_h).reshape(V1, RFH, DREAL)
    return cmat, v, w


def _table_body(cmr, vr, cmi, vi, wr, wi, outr_ref, outi_ref, qr, qi):
    i = pl.program_id(0)

    @pl.when(i == 0)
    def _():
        for j in range(V2):
            qr[pl.ds(j * V3, V3), :] = jnp.dot(
                cmr[...], vr[j], preferred_element_type=jnp.float32)
            qi[pl.ds(j * V3, V3), :] = jnp.dot(
                cmi[...], vi[j], preferred_element_type=jnp.float32)

    outr_ref[...] = jnp.dot(qr[...], wr[0], preferred_element_type=jnp.float32)
    outi_ref[...] = jnp.dot(qi[...], wi[0], preferred_element_type=jnp.float32)


def _build_tables(cmr, vr, wr, cmi, vi, wi):
    return pl.pallas_call(
        _table_body,
        grid=(V1,),
        in_specs=[
            pl.BlockSpec((V3, GH), lambda i: (0, 0)),
            pl.BlockSpec((V2, GH, RFH), lambda i: (0, 0, 0)),
            pl.BlockSpec((V3, GH), lambda i: (0, 0)),
            pl.BlockSpec((V2, GH, RFH), lambda i: (0, 0, 0)),
            pl.BlockSpec((1, RFH, DREAL), lambda i: (i, 0, 0)),
            pl.BlockSpec((1, RFH, DREAL), lambda i: (i, 0, 0)),
        ],
        out_specs=[
            pl.BlockSpec((ROWS_PER_I, DREAL), lambda i: (i, 0)),
            pl.BlockSpec((ROWS_PER_I, DREAL), lambda i: (i, 0)),
        ],
        out_shape=[
            jax.ShapeDtypeStruct((NROWS, DREAL), jnp.float32),
            jax.ShapeDtypeStruct((NROWS, DREAL), jnp.float32),
        ],
        scratch_shapes=[
            pltpu.VMEM((ROWS_PER_I, RFH), jnp.float32),
            pltpu.VMEM((ROWS_PER_I, RFH), jnp.float32),
        ],
    )(cmr, vr, cmi, vi, wr, wi)


SPAD = 56
CHUNK = 128


def _make_gather(n_tokens):
    info = plsc.get_sparse_core_info()
    nw = info.num_cores * info.num_subcores
    per_w = n_tokens // nw
    n_chunks = per_w // CHUNK
    assert per_w * nw == n_tokens and n_chunks * CHUNK == per_w
    assert n_chunks % 2 == 0
    mesh = plsc.VectorSubcoreMesh(core_axis_name="c", subcore_axis_name="s")

    @functools.partial(
        pl.kernel,
        mesh=mesh,
        out_type=[
            jax.ShapeDtypeStruct((n_tokens, DREAL), jnp.float32),
            jax.ShapeDtypeStruct((n_tokens, DREAL), jnp.float32),
        ],
        scratch_types=[
            pltpu.VMEM((per_w,), jnp.int32),
            pltpu.VMEM((2, CHUNK, DREAL), jnp.float32),
            pltpu.VMEM((2, CHUNK, DREAL), jnp.float32),
            pltpu.SemaphoreType.DMA,
            pltpu.SemaphoreType.DMA,
            pltpu.SemaphoreType.DMA,
            pltpu.SemaphoreType.DMA,
        ],
    )
    def gather_k(ids_hbm, tr_hbm, ti_hbm, outr_hbm, outi_hbm,
                 idx_v, rowsr_v, rowsi_v, sg0, sg1, ss0, ss1):
        sem_g = (sg0, sg1)
        sem_s = (ss0, ss1)
        wid = lax.axis_index("s") * info.num_cores + lax.axis_index("c")
        base = pl.multiple_of(wid * per_w, per_w)
        pltpu.sync_copy(ids_hbm.at[pl.ds(base, per_w)], idx_v)

        def mod_slice(t, carry):
            off = pl.multiple_of(t * 16, 16)
            idx_v[pl.ds(off, 16)] = lax.rem(idx_v[pl.ds(off, 16)], NROWS)
            return carry

        def g_copies(g, b):
            off = pl.multiple_of(g * CHUNK, CHUNK)
            sl = idx_v.at[pl.ds(off, CHUNK)]
            return (pltpu.make_async_copy(tr_hbm.at[sl], rowsr_v.at[b], sem_g[b]),
                    pltpu.make_async_copy(ti_hbm.at[sl], rowsi_v.at[b], sem_g[b]))

        def s_copies(g, b):
            off = pl.multiple_of(g * CHUNK, CHUNK)
            dst = pl.ds(base + off, CHUNK)
            return (pltpu.make_async_copy(rowsr_v.at[b], outr_hbm.at[dst], sem_s[b]),
                    pltpu.make_async_copy(rowsi_v.at[b], outi_hbm.at[dst], sem_s[b]))

        def start(copies):
            for c in copies:
                c.start()

        def wait(copies):
            for c in copies:
                c.wait()

        lax.fori_loop(0, CHUNK // 16, mod_slice, 0)
        start(g_copies(0, 0))
        lax.fori_loop(CHUNK // 16, per_w // 16, mod_slice, 0)

        def body(t, carry):
            for b in range(2):
                g = t * 2 + b
                wait(g_copies(g, b))
                start(s_copies(g, b))

                @pl.when(g >= 1)
                def _():
                    wait(s_copies(g - 1, 1 - b))

                @pl.when(g + 1 < n_chunks)
                def _():
                    start(g_copies(g + 1, 1 - b))

            return carry

        lax.fori_loop(0, n_chunks // 2, body, 0)
        wait(s_copies(n_chunks - 1, (n_chunks - 1) % 2))

    return gather_k


BBLK = 32


def _retile_body(r_ref, i_ref, out_ref):
    seq = out_ref.shape[1]
    r = r_ref[...].reshape(BBLK, SPAD, DREAL)
    i = i_ref[...].reshape(BBLK, SPAD, DREAL)
    out_ref[..., :DREAL] = r[:, :seq, :]
    out_ref[..., DREAL:] = i[:, :seq, :]


def _retile(plane_r, plane_i, batch, seq):
    return pl.pallas_call(
        _retile_body,
        grid=(batch // BBLK,),
        in_specs=[
            pl.BlockSpec((BBLK * SPAD, DREAL), lambda g: (g, 0)),
            pl.BlockSpec((BBLK * SPAD, DREAL), lambda g: (g, 0)),
        ],
        out_specs=pl.BlockSpec((BBLK, seq, DOUT), lambda g: (g, 0, 0)),
        out_shape=jax.ShapeDtypeStruct((batch, seq, DOUT), jnp.float32),
    )(plane_r, plane_i)


def kernel(input_ids, cr0, cr1, cr2, ci0, ci1, ci2):
    B, S = input_ids.shape
    ids = jnp.pad(input_ids.astype(jnp.int32), ((0, 0), (0, SPAD - S)))
    ids = ids.reshape(-1)
    cmr, vr, wr = _prep_branch(cr0, cr1, cr2)
    cmi, vi, wi = _prep_branch(ci0, ci1, ci2)
    table_r, table_i = _build_tables(cmr, vr, wr, cmi, vi, wi)
    plane_r, plane_i = _make_gather(B * SPAD)(ids, table_r, table_i)
    return _retile(plane_r, plane_i, B, S)


# speedup vs baseline: 3.4773x; 3.1358x over previous
import functools

import jax
import jax.numpy as jnp
from jax import lax
from jax.experimental import pallas as pl
from jax.experimental.pallas import tpu as pltpu
from jax.experimental.pallas import tpu_sc as plsc

V1, V2, V3 = 20, 40, 40
D1, D2, D3 = 4, 4, 8
RR = 4
NROWS = V1 * V2 * V3
DREAL = D1 * D2 * D3
DOUT = 2 * DREAL
ROWS_PER_I = V2 * V3
GH = RR * D3
RFH = RR * D2 * D3


def _prep_branch(c0, c1, c2):
    a = c0[0]
    b = c1
    c = c2[..., 0]
    eye_f = jnp.eye(D2, dtype=jnp.float32)
    eye_h = jnp.eye(D3, dtype=jnp.float32)
    cmat = c.transpose(1, 0, 2).reshape(V3, GH)
    v = jnp.einsum('rjfg,hq->jghrfq', b, eye_h).reshape(V2, GH, RFH)
    w = jnp.einsum('idr,fp,hq->irfhdpq', a, eye_f, eye_h).reshape(V1, RFH, DREAL)
    return cmat, v, w


def _table_body(cmr, vr, cmi, vi, wr, wi, outr_ref, outi_ref, qr, qi):
    i = pl.program_id(0)

    @pl.when(i == 0)
    def _():
        for j in range(V2):
            qr[pl.ds(j * V3, V3), :] = jnp.dot(
                cmr[...], vr[j], preferred_element_type=jnp.float32)
            qi[pl.ds(j * V3, V3), :] = jnp.dot(
                cmi[...], vi[j], preferred_element_type=jnp.float32)

    outr_ref[...] = jnp.dot(qr[...], wr[0], preferred_element_type=jnp.float32)
    outi_ref[...] = jnp.dot(qi[...], wi[0], preferred_element_type=jnp.float32)


def _build_tables(cmr, vr, wr, cmi, vi, wi):
    return pl.pallas_call(
        _table_body,
        grid=(V1,),
        in_specs=[
            pl.BlockSpec((V3, GH), lambda i: (0, 0)),
            pl.BlockSpec((V2, GH, RFH), lambda i: (0, 0, 0)),
            pl.BlockSpec((V3, GH), lambda i: (0, 0)),
            pl.BlockSpec((V2, GH, RFH), lambda i: (0, 0, 0)),
            pl.BlockSpec((1, RFH, DREAL), lambda i: (i, 0, 0)),
            pl.BlockSpec((1, RFH, DREAL), lambda i: (i, 0, 0)),
        ],
        out_specs=[
            pl.BlockSpec((ROWS_PER_I, DREAL), lambda i: (i, 0)),
            pl.BlockSpec((ROWS_PER_I, DREAL), lambda i: (i, 0)),
        ],
        out_shape=[
            jax.ShapeDtypeStruct((NROWS, DREAL), jnp.float32),
            jax.ShapeDtypeStruct((NROWS, DREAL), jnp.float32),
        ],
        scratch_shapes=[
            pltpu.VMEM((ROWS_PER_I, RFH), jnp.float32),
            pltpu.VMEM((ROWS_PER_I, RFH), jnp.float32),
        ],
    )(cmr, vr, cmi, vi, wr, wi)


SPAD = 56
CHUNK = 128


def _make_gather(n_tokens):
    info = plsc.get_sparse_core_info()
    nw = info.num_cores * info.num_subcores
    per_w = n_tokens // nw
    n_chunks = per_w // CHUNK
    assert per_w * nw == n_tokens and n_chunks * CHUNK == per_w
    assert n_chunks % 2 == 0
    mesh = plsc.VectorSubcoreMesh(core_axis_name="c", subcore_axis_name="s")

    @functools.partial(
        pl.kernel,
        mesh=mesh,
        out_type=[
            jax.ShapeDtypeStruct((n_tokens, DREAL), jnp.float32),
            jax.ShapeDtypeStruct((n_tokens, DREAL), jnp.float32),
        ],
        scratch_types=[
            pltpu.VMEM((per_w,), jnp.int32),
            pltpu.VMEM((2, CHUNK, DREAL), jnp.float32),
            pltpu.VMEM((2, CHUNK, DREAL), jnp.float32),
            pltpu.SemaphoreType.DMA,
            pltpu.SemaphoreType.DMA,
            pltpu.SemaphoreType.DMA,
            pltpu.SemaphoreType.DMA,
        ],
    )
    def gather_k(ids_hbm, tr_hbm, ti_hbm, outr_hbm, outi_hbm,
                 idx_v, rowsr_v, rowsi_v, sg0, sg1, ss0, ss1):
        sem_g = (sg0, sg1)
        sem_s = (ss0, ss1)
        wid = lax.axis_index("s") * info.num_cores + lax.axis_index("c")
        base = pl.multiple_of(wid * per_w, per_w)
        pltpu.sync_copy(ids_hbm.at[pl.ds(base, per_w)], idx_v)

        def mod_slice(t, carry):
            off = pl.multiple_of(t * 16, 16)
            idx_v[pl.ds(off, 16)] = lax.rem(idx_v[pl.ds(off, 16)], NROWS)
            return carry

        def g_copies(g, b):
            off = pl.multiple_of(g * CHUNK, CHUNK)
            sl = idx_v.at[pl.ds(off, CHUNK)]
            return (pltpu.make_async_copy(tr_hbm.at[sl], rowsr_v.at[b], sem_g[b]),
                    pltpu.make_async_copy(ti_hbm.at[sl], rowsi_v.at[b], sem_g[b]))

        def s_copies(g, b):
            off = pl.multiple_of(g * CHUNK, CHUNK)
            dst = pl.ds(base + off, CHUNK)
            return (pltpu.make_async_copy(rowsr_v.at[b], outr_hbm.at[dst], sem_s[b]),
                    pltpu.make_async_copy(rowsi_v.at[b], outi_hbm.at[dst], sem_s[b]))

        def start(copies):
            for c in copies:
                c.start()

        def wait(copies):
            for c in copies:
                c.wait()

        lax.fori_loop(0, CHUNK // 16, mod_slice, 0)
        start(g_copies(0, 0))
        lax.fori_loop(CHUNK // 16, per_w // 16, mod_slice, 0)

        def body(t, carry):
            for b in range(2):
                g = t * 2 + b
                wait(g_copies(g, b))
                start(s_copies(g, b))

                @pl.when(g >= 1)
                def _():
                    wait(s_copies(g - 1, 1 - b))

                @pl.when(g + 1 < n_chunks)
                def _():
                    start(g_copies(g + 1, 1 - b))

            return carry

        lax.fori_loop(0, n_chunks // 2, body, 0)
        wait(s_copies(n_chunks - 1, (n_chunks - 1) % 2))

    return gather_k


BBLK = 32


def _retile_body(r_ref, i_ref, out_ref):
    seq = out_ref.shape[1]
    r = r_ref[...].reshape(BBLK, SPAD, DREAL)
    i = i_ref[...].reshape(BBLK, SPAD, DREAL)
    out_ref[..., :DREAL] = r[:, :seq, :]
    out_ref[..., DREAL:] = i[:, :seq, :]


def _retile(plane_r, plane_i, batch, seq):
    return pl.pallas_call(
        _retile_body,
        grid=(batch // BBLK,),
        in_specs=[
            pl.BlockSpec((BBLK * SPAD, DREAL), lambda g: (g, 0)),
            pl.BlockSpec((BBLK * SPAD, DREAL), lambda g: (g, 0)),
        ],
        out_specs=pl.BlockSpec((BBLK, seq, DOUT), lambda g: (g, 0, 0)),
        out_shape=jax.ShapeDtypeStruct((batch, seq, DOUT), jnp.float32),
    )(plane_r, plane_i)


def kernel(input_ids, cr0, cr1, cr2, ci0, ci1, ci2):
    B, S = input_ids.shape
    ids32 = input_ids.astype(jnp.int32)
    ids = jnp.concatenate([ids32, ids32[:, :SPAD - S]], axis=1).reshape(-1)
    cmr, vr, wr = _prep_branch(cr0, cr1, cr2)
    cmi, vi, wi = _prep_branch(ci0, ci1, ci2)
    table_r, table_i = _build_tables(cmr, vr, wr, cmi, vi, wi)
    plane_r, plane_i = _make_gather(B * SPAD)(ids, table_r, table_i)
    return _retile(plane_r, plane_i, B, S)


# speedup vs baseline: 4.5001x; 1.2942x over previous
import functools

import jax
import jax.numpy as jnp
from jax import lax
from jax.experimental import pallas as pl
from jax.experimental.pallas import tpu as pltpu
from jax.experimental.pallas import tpu_sc as plsc

V1, V2, V3 = 20, 40, 40
D1, D2, D3 = 4, 4, 8
RR = 4
NROWS = V1 * V2 * V3
DREAL = D1 * D2 * D3
DOUT = 2 * DREAL
ROWS_PER_I = V2 * V3
GH = RR * D3
RFH = RR * D2 * D3


def _prep_branch(c0, c1, c2):
    a = c0[0]
    b = c1
    c = c2[..., 0]
    eye_f = jnp.eye(D2, dtype=jnp.float32)
    eye_h = jnp.eye(D3, dtype=jnp.float32)
    cmat = c.transpose(1, 0, 2).reshape(V3, GH)
    v = jnp.einsum('rjfg,hq->jghrfq', b, eye_h).reshape(V2, GH, RFH)
    w = jnp.einsum('idr,fp,hq->irfhdpq', a, eye_f, eye_h).reshape(V1, RFH, DREAL)
    return cmat, v, w


def _table_body(cmr, vr, cmi, vi, wr, wi, out_ref, qr, qi):
    i = pl.program_id(0)

    @pl.when(i == 0)
    def _():
        for j in range(V2):
            qr[pl.ds(j * V3, V3), :] = jnp.dot(
                cmr[...], vr[j], preferred_element_type=jnp.float32)
            qi[pl.ds(j * V3, V3), :] = jnp.dot(
                cmi[...], vi[j], preferred_element_type=jnp.float32)

    out_ref[0] = jnp.dot(qr[...], wr[0], preferred_element_type=jnp.float32)
    out_ref[1] = jnp.dot(qi[...], wi[0], preferred_element_type=jnp.float32)


def _build_tables(cmr, vr, wr, cmi, vi, wi):
    return pl.pallas_call(
        _table_body,
        grid=(V1,),
        in_specs=[
            pl.BlockSpec((V3, GH), lambda i: (0, 0)),
            pl.BlockSpec((V2, GH, RFH), lambda i: (0, 0, 0)),
            pl.BlockSpec((V3, GH), lambda i: (0, 0)),
            pl.BlockSpec((V2, GH, RFH), lambda i: (0, 0, 0)),
            pl.BlockSpec((1, RFH, DREAL), lambda i: (i, 0, 0)),
            pl.BlockSpec((1, RFH, DREAL), lambda i: (i, 0, 0)),
        ],
        out_specs=pl.BlockSpec((2, ROWS_PER_I, DREAL), lambda i: (0, i, 0)),
        out_shape=jax.ShapeDtypeStruct((2, NROWS, DREAL), jnp.float32),
        scratch_shapes=[
            pltpu.VMEM((ROWS_PER_I, RFH), jnp.float32),
            pltpu.VMEM((ROWS_PER_I, RFH), jnp.float32),
        ],
    )(cmr, vr, cmi, vi, wr, wi)


CHUNK = 128


def _make_gather(n_tokens):
    info = plsc.get_sparse_core_info()
    nw = info.num_cores * info.num_subcores
    per_w = n_tokens // nw
    n_chunks = per_w // CHUNK
    assert per_w * nw == n_tokens and n_chunks * CHUNK == per_w
    assert n_chunks % 2 == 0
    mesh = plsc.VectorSubcoreMesh(core_axis_name="c", subcore_axis_name="s")

    @functools.partial(
        pl.kernel,
        mesh=mesh,
        out_type=[
            jax.ShapeDtypeStruct((n_tokens, DREAL), jnp.float32),
            jax.ShapeDtypeStruct((n_tokens, DREAL), jnp.float32),
        ],
        scratch_types=[
            pltpu.VMEM((per_w,), jnp.int32),
            pltpu.VMEM((2, CHUNK, DREAL), jnp.float32),
            pltpu.VMEM((2, CHUNK, DREAL), jnp.float32),
            pltpu.SemaphoreType.DMA,
            pltpu.SemaphoreType.DMA,
            pltpu.SemaphoreType.DMA,
            pltpu.SemaphoreType.DMA,
        ],
    )
    def gather_k(ids_hbm, tables_hbm, outr_hbm, outi_hbm,
                 idx_v, rowsr_v, rowsi_v, sg0, sg1, ss0, ss1):
        sem_g = (sg0, sg1)
        sem_s = (ss0, ss1)
        wid = lax.axis_index("s") * info.num_cores + lax.axis_index("c")
        base = pl.multiple_of(wid * per_w, per_w)
        pltpu.sync_copy(ids_hbm.at[pl.ds(base, per_w)], idx_v)

        def mod_slice(t, carry):
            off = pl.multiple_of(t * 16, 16)
            idx_v[pl.ds(off, 16)] = lax.rem(idx_v[pl.ds(off, 16)], NROWS)
            return carry

        def g_copies(g, b):
            off = pl.multiple_of(g * CHUNK, CHUNK)
            sl = idx_v.at[pl.ds(off, CHUNK)]
            return (pltpu.make_async_copy(tables_hbm.at[0].at[sl], rowsr_v.at[b],
                                          sem_g[b]),
                    pltpu.make_async_copy(tables_hbm.at[1].at[sl], rowsi_v.at[b],
                                          sem_g[b]))

        def s_copies(g, b):
            off = pl.multiple_of(g * CHUNK, CHUNK)
            dst = pl.ds(base + off, CHUNK)
            return (pltpu.make_async_copy(rowsr_v.at[b], outr_hbm.at[dst], sem_s[b]),
                    pltpu.make_async_copy(rowsi_v.at[b], outi_hbm.at[dst], sem_s[b]))

        def start(copies):
            for c in copies:
                c.start()

        def wait(copies):
            for c in copies:
                c.wait()

        lax.fori_loop(0, CHUNK // 16, mod_slice, 0)
        start(g_copies(0, 0))
        lax.fori_loop(CHUNK // 16, per_w // 16, mod_slice, 0)

        def body(t, carry):
            for b in range(2):
                g = t * 2 + b
                wait(g_copies(g, b))
                start(s_copies(g, b))

                @pl.when(g >= 1)
                def _():
                    wait(s_copies(g - 1, 1 - b))

                @pl.when(g + 1 < n_chunks)
                def _():
                    start(g_copies(g + 1, 1 - b))

            return carry

        lax.fori_loop(0, n_chunks // 2, body, 0)
        wait(s_copies(n_chunks - 1, (n_chunks - 1) % 2))

    return gather_k


BCHUNK = 1024


def _merge_body(r_ref, i_ref, out_ref):
    out_ref[0, :, :DREAL] = r_ref[0]
    out_ref[0, :, DREAL:] = i_ref[0]


def _merge(plane_r, plane_i, batch, seq):
    return pl.pallas_call(
        _merge_body,
        grid=(seq, batch // BCHUNK),
        in_specs=[
            pl.BlockSpec((1, BCHUNK, DREAL), lambda s, c: (s, c, 0)),
            pl.BlockSpec((1, BCHUNK, DREAL), lambda s, c: (s, c, 0)),
        ],
        out_specs=pl.BlockSpec((1, BCHUNK, DOUT), lambda s, c: (s, c, 0)),
        out_shape=jax.ShapeDtypeStruct((seq, batch, DOUT), jnp.float32),
    )(plane_r.reshape(seq, batch, DREAL), plane_i.reshape(seq, batch, DREAL))


def kernel(input_ids, cr0, cr1, cr2, ci0, ci1, ci2):
    B, S = input_ids.shape
    ids = input_ids.astype(jnp.int32).T.reshape(-1)
    cmr, vr, wr = _prep_branch(cr0, cr1, cr2)
    cmi, vi, wi = _prep_branch(ci0, ci1, ci2)
    tables = _build_tables(cmr, vr, wr, cmi, vi, wi)
    plane_r, plane_i = _make_gather(B * S)(ids, tables)
    out_t = _merge(plane_r, plane_i, B, S)
    return out_t.transpose(1, 0, 2)


# speedup vs baseline: 5.1763x; 1.1502x over previous
import functools

import jax
import jax.numpy as jnp
from jax import lax
from jax.experimental import pallas as pl
from jax.experimental.pallas import tpu as pltpu
from jax.experimental.pallas import tpu_sc as plsc

V1, V2, V3 = 20, 40, 40
D1, D2, D3 = 4, 4, 8
RR = 4
NROWS = V1 * V2 * V3
DREAL = D1 * D2 * D3
DOUT = 2 * DREAL
ROWS_PER_I = V2 * V3
GH = RR * D3
RFH = RR * D2 * D3


def _prep_branch(c0, c1, c2):
    a = c0[0]
    b = c1
    c = c2[..., 0]
    eye_f = jnp.eye(D2, dtype=jnp.float32)
    eye_h = jnp.eye(D3, dtype=jnp.float32)
    cmat = c.transpose(1, 0, 2).reshape(V3, GH)
    v = jnp.einsum('rjfg,hq->jghrfq', b, eye_h).reshape(V2, GH, RFH)
    w = jnp.einsum('idr,fp,hq->irfhdpq', a, eye_f, eye_h).reshape(V1, RFH, DREAL)
    return cmat, v, w


def _table_body(cmr, vr, cmi, vi, wr, wi, out_ref, qr, qi):
    i = pl.program_id(0)

    @pl.when(i == 0)
    def _():
        for j in range(V2):
            qr[pl.ds(j * V3, V3), :] = jnp.dot(
                cmr[...], vr[j], preferred_element_type=jnp.float32)
            qi[pl.ds(j * V3, V3), :] = jnp.dot(
                cmi[...], vi[j], preferred_element_type=jnp.float32)

    out_ref[0] = jnp.dot(qr[...], wr[0], preferred_element_type=jnp.float32)
    out_ref[1] = jnp.dot(qi[...], wi[0], preferred_element_type=jnp.float32)


def _build_tables(cmr, vr, wr, cmi, vi, wi):
    return pl.pallas_call(
        _table_body,
        grid=(V1,),
        in_specs=[
            pl.BlockSpec((V3, GH), lambda i: (0, 0)),
            pl.BlockSpec((V2, GH, RFH), lambda i: (0, 0, 0)),
            pl.BlockSpec((V3, GH), lambda i: (0, 0)),
            pl.BlockSpec((V2, GH, RFH), lambda i: (0, 0, 0)),
            pl.BlockSpec((1, RFH, DREAL), lambda i: (i, 0, 0)),
            pl.BlockSpec((1, RFH, DREAL), lambda i: (i, 0, 0)),
        ],
        out_specs=pl.BlockSpec((2, ROWS_PER_I, DREAL), lambda i: (0, i, 0)),
        out_shape=jax.ShapeDtypeStruct((2, NROWS, DREAL), jnp.float32),
        scratch_shapes=[
            pltpu.VMEM((ROWS_PER_I, RFH), jnp.float32),
            pltpu.VMEM((ROWS_PER_I, RFH), jnp.float32),
        ],
    )(cmr, vr, cmi, vi, wr, wi)


CHUNK = 128


def _make_gather(n_tokens):
    info = plsc.get_sparse_core_info()
    nw = info.num_cores * info.num_subcores
    per_w = n_tokens // nw
    n_chunks = per_w // CHUNK
    assert per_w * nw == n_tokens and n_chunks * CHUNK == per_w
    assert n_chunks % 2 == 0
    mesh = plsc.VectorSubcoreMesh(core_axis_name="c", subcore_axis_name="s")

    @functools.partial(
        pl.kernel,
        mesh=mesh,
        out_type=[
            jax.ShapeDtypeStruct((n_tokens, DREAL), jnp.float32),
            jax.ShapeDtypeStruct((n_tokens, DREAL), jnp.float32),
        ],
        scratch_types=[
            pltpu.VMEM((per_w,), jnp.int32),
            pltpu.VMEM((2, CHUNK, DREAL), jnp.float32),
            pltpu.VMEM((2, CHUNK, DREAL), jnp.float32),
            pltpu.SemaphoreType.DMA,
            pltpu.SemaphoreType.DMA,
            pltpu.SemaphoreType.DMA,
            pltpu.SemaphoreType.DMA,
        ],
    )
    def gather_k(ids_hbm, tables_hbm, outr_hbm, outi_hbm,
                 idx_v, rowsr_v, rowsi_v, sg0, sg1, ss0, ss1):
        sem_g = (sg0, sg1)
        sem_s = (ss0, ss1)
        wid = lax.axis_index("s") * info.num_cores + lax.axis_index("c")
        base = pl.multiple_of(wid * per_w, per_w)
        pltpu.sync_copy(ids_hbm.at[pl.ds(base, per_w)], idx_v)

        def mod_slice(t, carry):
            off = pl.multiple_of(t * 16, 16)
            idx_v[pl.ds(off, 16)] = lax.rem(idx_v[pl.ds(off, 16)], NROWS)
            return carry

        def g_copies(g, b):
            off = pl.multiple_of(g * CHUNK, CHUNK)
            sl = idx_v.at[pl.ds(off, CHUNK)]
            return (pltpu.make_async_copy(tables_hbm.at[0].at[sl], rowsr_v.at[b],
                                          sem_g[b]),
                    pltpu.make_async_copy(tables_hbm.at[1].at[sl], rowsi_v.at[b],
                                          sem_g[b]))

        def s_copies(g, b):
            off = pl.multiple_of(g * CHUNK, CHUNK)
            dst = pl.ds(base + off, CHUNK)
            return (pltpu.make_async_copy(rowsr_v.at[b], outr_hbm.at[dst], sem_s[b]),
                    pltpu.make_async_copy(rowsi_v.at[b], outi_hbm.at[dst], sem_s[b]))

        def start(copies):
            for c in copies:
                c.start()

        def wait(copies):
            for c in copies:
                c.wait()

        lax.fori_loop(0, CHUNK // 16, mod_slice, 0)
        start(g_copies(0, 0))
        lax.fori_loop(CHUNK // 16, per_w // 16, mod_slice, 0)

        def body(t, carry):
            for b in range(2):
                g = t * 2 + b
                wait(g_copies(g, b))
                start(s_copies(g, b))

                @pl.when(g >= 1)
                def _():
                    wait(s_copies(g - 1, 1 - b))

                @pl.when(g + 1 < n_chunks)
                def _():
                    start(g_copies(g + 1, 1 - b))

            return carry

        lax.fori_loop(0, n_chunks // 2, body, 0)
        wait(s_copies(n_chunks - 1, (n_chunks - 1) % 2))

    return gather_k


BCHUNK = 2048


def _merge_body(r_ref, i_ref, out_ref):
    out_ref[0, :, :DREAL] = r_ref[0]
    out_ref[0, :, DREAL:] = i_ref[0]


def _merge(plane_r, plane_i, batch, seq):
    return pl.pallas_call(
        _merge_body,
        grid=(seq, batch // BCHUNK),
        in_specs=[
            pl.BlockSpec((1, BCHUNK, DREAL), lambda s, c: (s, c, 0)),
            pl.BlockSpec((1, BCHUNK, DREAL), lambda s, c: (s, c, 0)),
        ],
        out_specs=pl.BlockSpec((1, BCHUNK, DOUT), lambda s, c: (s, c, 0)),
        out_shape=jax.ShapeDtypeStruct((seq, batch, DOUT), jnp.float32),
    )(plane_r.reshape(seq, batch, DREAL), plane_i.reshape(seq, batch, DREAL))


def kernel(input_ids, cr0, cr1, cr2, ci0, ci1, ci2):
    B, S = input_ids.shape
    ids = input_ids.astype(jnp.int32).T.reshape(-1)
    cmr, vr, wr = _prep_branch(cr0, cr1, cr2)
    cmi, vi, wi = _prep_branch(ci0, ci1, ci2)
    tables = _build_tables(cmr, vr, wr, cmi, vi, wi)
    plane_r, plane_i = _make_gather(B * S)(ids, tables)
    out_t = _merge(plane_r, plane_i, B, S)
    return out_t.transpose(1, 0, 2)


# speedup vs baseline: 5.3430x; 1.0322x over previous
import functools

import jax
import jax.numpy as jnp
from jax import lax
from jax.experimental import pallas as pl
from jax.experimental.pallas import tpu as pltpu
from jax.experimental.pallas import tpu_sc as plsc

V1, V2, V3 = 20, 40, 40
D1, D2, D3 = 4, 4, 8
RR = 4
NROWS = V1 * V2 * V3
DREAL = D1 * D2 * D3
DOUT = 2 * DREAL
ROWS_PER_I = V2 * V3
GH = RR * D3
RFH = RR * D2 * D3


def _prep_branch(c0, c1, c2):
    a = c0[0]
    b = c1
    c = c2[..., 0]
    eye_f = jnp.eye(D2, dtype=jnp.float32)
    eye_h = jnp.eye(D3, dtype=jnp.float32)
    cmat = c.transpose(1, 0, 2).reshape(V3, GH)
    v = jnp.einsum('rjfg,hq->jghrfq', b, eye_h).reshape(V2, GH, RFH)
    w = jnp.einsum('idr,fp,hq->irfhdpq', a, eye_f, eye_h).reshape(V1, RFH, DREAL)
    return cmat, v, w


def _table_body(cmr, vr, cmi, vi, wr, wi, out_ref, qr, qi):
    i = pl.program_id(0)

    @pl.when(i == 0)
    def _():
        for j in range(V2):
            qr[pl.ds(j * V3, V3), :] = jnp.dot(
                cmr[...], vr[j], preferred_element_type=jnp.float32)
            qi[pl.ds(j * V3, V3), :] = jnp.dot(
                cmi[...], vi[j], preferred_element_type=jnp.float32)

    out_ref[0] = jnp.dot(qr[...], wr[0], preferred_element_type=jnp.float32)
    out_ref[1] = jnp.dot(qi[...], wi[0], preferred_element_type=jnp.float32)


def _build_tables(cmr, vr, wr, cmi, vi, wi):
    return pl.pallas_call(
        _table_body,
        grid=(V1,),
        in_specs=[
            pl.BlockSpec((V3, GH), lambda i: (0, 0)),
            pl.BlockSpec((V2, GH, RFH), lambda i: (0, 0, 0)),
            pl.BlockSpec((V3, GH), lambda i: (0, 0)),
            pl.BlockSpec((V2, GH, RFH), lambda i: (0, 0, 0)),
            pl.BlockSpec((1, RFH, DREAL), lambda i: (i, 0, 0)),
            pl.BlockSpec((1, RFH, DREAL), lambda i: (i, 0, 0)),
        ],
        out_specs=pl.BlockSpec((2, ROWS_PER_I, DREAL), lambda i: (0, i, 0)),
        out_shape=jax.ShapeDtypeStruct((2, NROWS, DREAL), jnp.float32),
        scratch_shapes=[
            pltpu.VMEM((ROWS_PER_I, RFH), jnp.float32),
            pltpu.VMEM((ROWS_PER_I, RFH), jnp.float32),
        ],
    )(cmr, vr, cmi, vi, wr, wi)


CHUNK = 128


def _make_gather(n_tokens):
    info = plsc.get_sparse_core_info()
    nw = info.num_cores * info.num_subcores
    per_w = n_tokens // nw
    n_chunks = per_w // CHUNK
    assert per_w * nw == n_tokens and n_chunks * CHUNK == per_w
    assert n_chunks % 2 == 0
    mesh = plsc.VectorSubcoreMesh(core_axis_name="c", subcore_axis_name="s")

    @functools.partial(
        pl.kernel,
        mesh=mesh,
        out_type=[
            jax.ShapeDtypeStruct((n_tokens, DREAL), jnp.float32),
            jax.ShapeDtypeStruct((n_tokens, DREAL), jnp.float32),
        ],
        scratch_types=[
            pltpu.VMEM((per_w,), jnp.int32),
            pltpu.VMEM((2, CHUNK, DREAL), jnp.float32),
            pltpu.VMEM((2, CHUNK, DREAL), jnp.float32),
            pltpu.SemaphoreType.DMA,
            pltpu.SemaphoreType.DMA,
            pltpu.SemaphoreType.DMA,
            pltpu.SemaphoreType.DMA,
        ],
    )
    def gather_k(ids_hbm, tables_hbm, outr_hbm, outi_hbm,
                 idx_v, rowsr_v, rowsi_v, sg0, sg1, ss0, ss1):
        sem_g = (sg0, sg1)
        sem_s = (ss0, ss1)
        wid = lax.axis_index("s") * info.num_cores + lax.axis_index("c")
        base = pl.multiple_of(wid * per_w, per_w)
        pltpu.sync_copy(ids_hbm.at[pl.ds(base, per_w)], idx_v)

        def mod_slice(t, carry):
            off = pl.multiple_of(t * 16, 16)
            idx_v[pl.ds(off, 16)] = lax.rem(idx_v[pl.ds(off, 16)], NROWS)
            return carry

        def g_copies(g, b):
            off = pl.multiple_of(g * CHUNK, CHUNK)
            sl = idx_v.at[pl.ds(off, CHUNK)]
            return (pltpu.make_async_copy(tables_hbm.at[0].at[sl], rowsr_v.at[b],
                                          sem_g[b]),
                    pltpu.make_async_copy(tables_hbm.at[1].at[sl], rowsi_v.at[b],
                                          sem_g[b]))

        def s_copies(g, b):
            off = pl.multiple_of(g * CHUNK, CHUNK)
            dst = pl.ds(base + off, CHUNK)
            return (pltpu.make_async_copy(rowsr_v.at[b], outr_hbm.at[dst], sem_s[b]),
                    pltpu.make_async_copy(rowsi_v.at[b], outi_hbm.at[dst], sem_s[b]))

        def start(copies):
            for c in copies:
                c.start()

        def wait(copies):
            for c in copies:
                c.wait()

        lax.fori_loop(0, CHUNK // 16, mod_slice, 0)
        start(g_copies(0, 0))
        lax.fori_loop(CHUNK // 16, per_w // 16, mod_slice, 0)

        def body(t, carry):
            for b in range(2):
                g = t * 2 + b
                wait(g_copies(g, b))
                start(s_copies(g, b))

                @pl.when(g >= 1)
                def _():
                    wait(s_copies(g - 1, 1 - b))

                @pl.when(g + 1 < n_chunks)
                def _():
                    start(g_copies(g + 1, 1 - b))

            return carry

        lax.fori_loop(0, n_chunks // 2, body, 0)
        wait(s_copies(n_chunks - 1, (n_chunks - 1) % 2))

    return gather_k


BCHUNK = 4096


def _merge_body(r_ref, i_ref, out_ref):
    out_ref[0, :, :DREAL] = r_ref[0]
    out_ref[0, :, DREAL:] = i_ref[0]


def _merge(plane_r, plane_i, batch, seq):
    return pl.pallas_call(
        _merge_body,
        grid=(seq, batch // BCHUNK),
        in_specs=[
            pl.BlockSpec((1, BCHUNK, DREAL), lambda s, c: (s, c, 0)),
            pl.BlockSpec((1, BCHUNK, DREAL), lambda s, c: (s, c, 0)),
        ],
        out_specs=pl.BlockSpec((1, BCHUNK, DOUT), lambda s, c: (s, c, 0)),
        out_shape=jax.ShapeDtypeStruct((seq, batch, DOUT), jnp.float32),
    )(plane_r.reshape(seq, batch, DREAL), plane_i.reshape(seq, batch, DREAL))


def kernel(input_ids, cr0, cr1, cr2, ci0, ci1, ci2):
    B, S = input_ids.shape
    ids = input_ids.astype(jnp.int32).T.reshape(-1)
    cmr, vr, wr = _prep_branch(cr0, cr1, cr2)
    cmi, vi, wi = _prep_branch(ci0, ci1, ci2)
    tables = _build_tables(cmr, vr, wr, cmi, vi, wi)
    plane_r, plane_i = _make_gather(B * S)(ids, tables)
    out_t = _merge(plane_r, plane_i, B, S)
    return out_t.transpose(1, 0, 2)


# speedup vs baseline: 5.4302x; 1.0163x over previous
import functools

import jax
import jax.numpy as jnp
from jax import lax
from jax.experimental import pallas as pl
from jax.experimental.pallas import tpu as pltpu
from jax.experimental.pallas import tpu_sc as plsc

V1, V2, V3 = 20, 40, 40
D1, D2, D3 = 4, 4, 8
RR = 4
NROWS = V1 * V2 * V3
DREAL = D1 * D2 * D3
DOUT = 2 * DREAL
ROWS_PER_I = V2 * V3
GH = RR * D3
RFH = RR * D2 * D3


def _prep_branch(c0, c1, c2):
    a = c0[0]
    b = c1
    c = c2[..., 0]
    eye_f = jnp.eye(D2, dtype=jnp.float32)
    eye_h = jnp.eye(D3, dtype=jnp.float32)
    cmat = c.transpose(1, 0, 2).reshape(V3, GH)
    v = jnp.einsum('rjfg,hq->jghrfq', b, eye_h).reshape(V2, GH, RFH)
    w = jnp.einsum('idr,fp,hq->irfhdpq', a, eye_f, eye_h).reshape(V1, RFH, DREAL)
    return cmat, v, w


def _table_body(cmr, vr, cmi, vi, wr, wi, out_ref, qr, qi):
    i = pl.program_id(0)

    @pl.when(i == 0)
    def _():
        for j in range(V2):
            qr[pl.ds(j * V3, V3), :] = jnp.dot(
                cmr[...], vr[j], preferred_element_type=jnp.float32)
            qi[pl.ds(j * V3, V3), :] = jnp.dot(
                cmi[...], vi[j], preferred_element_type=jnp.float32)

    out_ref[0] = jnp.dot(qr[...], wr[0], preferred_element_type=jnp.float32)
    out_ref[1] = jnp.dot(qi[...], wi[0], preferred_element_type=jnp.float32)


def _build_tables(cmr, vr, wr, cmi, vi, wi):
    return pl.pallas_call(
        _table_body,
        grid=(V1,),
        in_specs=[
            pl.BlockSpec((V3, GH), lambda i: (0, 0)),
            pl.BlockSpec((V2, GH, RFH), lambda i: (0, 0, 0)),
            pl.BlockSpec((V3, GH), lambda i: (0, 0)),
            pl.BlockSpec((V2, GH, RFH), lambda i: (0, 0, 0)),
            pl.BlockSpec((1, RFH, DREAL), lambda i: (i, 0, 0)),
            pl.BlockSpec((1, RFH, DREAL), lambda i: (i, 0, 0)),
        ],
        out_specs=pl.BlockSpec((2, ROWS_PER_I, DREAL), lambda i: (0, i, 0)),
        out_shape=jax.ShapeDtypeStruct((2, NROWS, DREAL), jnp.float32),
        scratch_shapes=[
            pltpu.VMEM((ROWS_PER_I, RFH), jnp.float32),
            pltpu.VMEM((ROWS_PER_I, RFH), jnp.float32),
        ],
    )(cmr, vr, cmi, vi, wr, wi)


CHUNK = 128


def _make_gather(n_tokens):
    info = plsc.get_sparse_core_info()
    nw = info.num_cores * info.num_subcores
    per_w = n_tokens // nw
    n_chunks = per_w // CHUNK
    assert per_w * nw == n_tokens and n_chunks * CHUNK == per_w
    assert n_chunks % 2 == 0
    mesh = plsc.VectorSubcoreMesh(core_axis_name="c", subcore_axis_name="s")

    @functools.partial(
        pl.kernel,
        mesh=mesh,
        out_type=[
            jax.ShapeDtypeStruct((n_tokens, DREAL), jnp.float32),
            jax.ShapeDtypeStruct((n_tokens, DREAL), jnp.float32),
        ],
        scratch_types=[
            pltpu.VMEM((per_w,), jnp.int32),
            pltpu.VMEM((2, CHUNK, DREAL), jnp.float32),
            pltpu.VMEM((2, CHUNK, DREAL), jnp.float32),
            pltpu.SemaphoreType.DMA,
            pltpu.SemaphoreType.DMA,
            pltpu.SemaphoreType.DMA,
            pltpu.SemaphoreType.DMA,
        ],
    )
    def gather_k(ids_hbm, tables_hbm, outr_hbm, outi_hbm,
                 idx_v, rowsr_v, rowsi_v, sg0, sg1, ss0, ss1):
        sem_g = (sg0, sg1)
        sem_s = (ss0, ss1)
        wid = lax.axis_index("s") * info.num_cores + lax.axis_index("c")
        base = pl.multiple_of(wid * per_w, per_w)
        pltpu.sync_copy(ids_hbm.at[pl.ds(base, per_w)], idx_v)

        def mod_slice(t, carry):
            off = pl.multiple_of(t * 16, 16)
            idx_v[pl.ds(off, 16)] = lax.rem(idx_v[pl.ds(off, 16)], NROWS)
            return carry

        def g_copies(g, b):
            off = pl.multiple_of(g * CHUNK, CHUNK)
            sl = idx_v.at[pl.ds(off, CHUNK)]
            return (pltpu.make_async_copy(tables_hbm.at[0].at[sl], rowsr_v.at[b],
                                          sem_g[b]),
                    pltpu.make_async_copy(tables_hbm.at[1].at[sl], rowsi_v.at[b],
                                          sem_g[b]))

        def s_copies(g, b):
            off = pl.multiple_of(g * CHUNK, CHUNK)
            dst = pl.ds(base + off, CHUNK)
            return (pltpu.make_async_copy(rowsr_v.at[b], outr_hbm.at[dst], sem_s[b]),
                    pltpu.make_async_copy(rowsi_v.at[b], outi_hbm.at[dst], sem_s[b]))

        def start(copies):
            for c in copies:
                c.start()

        def wait(copies):
            for c in copies:
                c.wait()

        lax.fori_loop(0, CHUNK // 16, mod_slice, 0)
        start(g_copies(0, 0))
        lax.fori_loop(CHUNK // 16, per_w // 16, mod_slice, 0)

        def body(t, carry):
            for b in range(2):
                g = t * 2 + b
                wait(g_copies(g, b))
                start(s_copies(g, b))

                @pl.when(g >= 1)
                def _():
                    wait(s_copies(g - 1, 1 - b))

                @pl.when(g + 1 < n_chunks)
                def _():
                    start(g_copies(g + 1, 1 - b))

            return carry

        lax.fori_loop(0, n_chunks // 2, body, 0)
        wait(s_copies(n_chunks - 1, (n_chunks - 1) % 2))

    return gather_k


SSPLIT = 24


def _merge_body(r_ref, i_ref, out_ref):
    out_ref[0, :, :DREAL] = r_ref[0]
    out_ref[0, :, DREAL:] = i_ref[0]


def _merge_body_acc(r_ref, i_ref, acc_ref, out_ref):
    del acc_ref
    out_ref[0, :, :DREAL] = r_ref[0]
    out_ref[0, :, DREAL:] = i_ref[0]


def _merge_first(plane_r, plane_i, batch, seq, s_n):
    return pl.pallas_call(
        _merge_body,
        grid=(s_n,),
        in_specs=[
            pl.BlockSpec((1, batch, DREAL), lambda s: (s, 0, 0)),
            pl.BlockSpec((1, batch, DREAL), lambda s: (s, 0, 0)),
        ],
        out_specs=pl.BlockSpec((1, batch, DOUT), lambda s: (s, 0, 0)),
        out_shape=jax.ShapeDtypeStruct((seq, batch, DOUT), jnp.float32),
    )(plane_r.reshape(s_n, batch, DREAL), plane_i.reshape(s_n, batch, DREAL))


def _merge_second(plane_r, plane_i, acc, batch, seq, s0):
    s_n = seq - s0
    return pl.pallas_call(
        _merge_body_acc,
        grid=(s_n,),
        in_specs=[
            pl.BlockSpec((1, batch, DREAL), lambda s: (s, 0, 0)),
            pl.BlockSpec((1, batch, DREAL), lambda s: (s, 0, 0)),
            pl.BlockSpec(memory_space=pl.ANY),
        ],
        out_specs=pl.BlockSpec((1, batch, DOUT), lambda s: (s0 + s, 0, 0)),
        out_shape=jax.ShapeDtypeStruct((seq, batch, DOUT), jnp.float32),
        input_output_aliases={2: 0},
    )(plane_r.reshape(s_n, batch, DREAL), plane_i.reshape(s_n, batch, DREAL),
      acc)


def kernel(input_ids, cr0, cr1, cr2, ci0, ci1, ci2):
    B, S = input_ids.shape
    ids = input_ids.astype(jnp.int32).T.reshape(-1)
    cmr, vr, wr = _prep_branch(cr0, cr1, cr2)
    cmi, vi, wi = _prep_branch(ci0, ci1, ci2)
    tables = _build_tables(cmr, vr, wr, cmi, vi, wi)
    n1 = SSPLIT * B
    pr1, pi1 = _make_gather(n1)(ids[:n1], tables)
    pr2, pi2 = _make_gather(B * S - n1)(ids[n1:], tables)
    out_t = _merge_first(pr1, pi1, B, S, SSPLIT)
    out_t = _merge_second(pr2, pi2, out_t, B, S, SSPLIT)
    return out_t.transpose(1, 0, 2)
